# baseline (speedup 1.0000x reference)
import functools

import jax
import jax.numpy as jnp
from jax import lax
from jax.experimental import pallas as pl
from jax.experimental.pallas import tpu as pltpu
from jax.experimental.pallas import tpu_sc as plsc

B, S, D = 2, 2048, 768
E, U1, U2, TOP_K = 8, 768, 768, 2


def _gate_kernel(x_ref, wa_ref, wg_ref, bg_ref, glog_ref):
    x = x_ref[0]
    wa = wa_ref[...]
    logits = jnp.sum(x * wa, axis=1, keepdims=True)
    m = jnp.max(logits)
    e = jnp.exp(logits - m)
    scores = e / jnp.sum(e)
    attn = jnp.sum(x * scores, axis=0, keepdims=True)
    glog = jnp.dot(attn, wg_ref[...],
                   preferred_element_type=jnp.float32) + bg_ref[...]
    b = pl.program_id(0)
    glog_ref[pl.ds(b, 1), :] = glog


def _route_kernel(glog_hbm, idx_hbm, w_hbm, g_v, idx_v, w_v):
    cid = lax.axis_index("c")
    sid = lax.axis_index("s")

    @pl.when((cid == 0) & (sid == 0))
    def _():
        pltpu.sync_copy(glog_hbm, g_v)
        l = g_v[...]
        lane = lax.iota(jnp.int32, 16)

        def vec(x):
            return jnp.broadcast_to(x, (16,))

        res = []
        for b in range(B):
            mb = (lane >= b * E) & (lane < (b + 1) * E)
            neg = jnp.float32(-3.4e38)
            m = vec(jnp.max(jnp.where(mb, l, neg)))
            eb = jnp.where(mb, jnp.exp(l - m), 0.0)
            z = vec(jnp.sum(eb))
            gs = jnp.where(mb, eb / z, -1.0)
            v1 = vec(jnp.max(gs))
            i1 = vec(plsc.all_reduce_ffs(gs == v1))
            g2 = jnp.where(lane == i1, -1.0, gs)
            v2 = vec(jnp.max(g2))
            i2 = vec(plsc.all_reduce_ffs(g2 == v2))
            s = v1 + v2 + 1e-9
            res.append((i1 - b * E, i2 - b * E, v1 / s, v2 / s))
        idx_vec = jnp.zeros((16,), jnp.int32)
        w_vec = jnp.zeros((16,), jnp.float32)
        for b in range(B):
            i1, i2, w1, w2 = res[b]
            idx_vec = jnp.where(lane == 2 * b, i1, idx_vec)
            idx_vec = jnp.where(lane == 2 * b + 1, i2, idx_vec)
            w_vec = jnp.where(lane == 2 * b, w1, w_vec)
            w_vec = jnp.where(lane == 2 * b + 1, w2, w_vec)
        idx_v[...] = idx_vec
        w_v[...] = w_vec
        pltpu.sync_copy(idx_v, idx_hbm)
        pltpu.sync_copy(w_v, w_hbm)


def _gelu_exact(x):
    return x * 0.5 * (1.0 + lax.erf(x * 0.7071067811865476))


S_BLK = 2048


def _ffn_kernel(idx_ref, w_ref, x_ref, w1_ref, b1_ref, w2_ref, b2_ref,
                out_ref):
    del idx_ref
    b = pl.program_id(0)
    k = pl.program_id(2)
    x = x_ref[0]
    h = jnp.dot(x, w1_ref[0], preferred_element_type=jnp.float32)
    h = _gelu_exact(h + b1_ref[0])
    o = jnp.dot(h, w2_ref[0], preferred_element_type=jnp.float32)
    o = _gelu_exact(o + b2_ref[0])
    w = w_ref[2 * b + k]

    @pl.when(k == 0)
    def _():
        out_ref[0] = w * o

    @pl.when(k == 1)
    def _():
        out_ref[0] += w * o


@jax.jit
def kernel(inputs, W_attn, b_attn, W_gate, b_gate, W1, b1, W2, b2):
    del b_attn
    wa_t = W_attn.reshape(1, D)
    bg = b_gate.reshape(1, E)

    glog = pl.pallas_call(
        _gate_kernel,
        grid=(B,),
        in_specs=[
            pl.BlockSpec((1, S, D), lambda b: (b, 0, 0)),
            pl.BlockSpec((1, D), lambda b: (0, 0)),
            pl.BlockSpec((D, E), lambda b: (0, 0)),
            pl.BlockSpec((1, E), lambda b: (0, 0)),
        ],
        out_specs=pl.BlockSpec((B, E), lambda b: (0, 0)),
        out_shape=jax.ShapeDtypeStruct((B, E), jnp.float32),
    )(inputs, wa_t, W_gate, bg)

    route = functools.partial(
        pl.kernel,
        mesh=plsc.VectorSubcoreMesh(core_axis_name="c", subcore_axis_name="s"),
        out_type=[
            jax.ShapeDtypeStruct((16,), jnp.int32),
            jax.ShapeDtypeStruct((16,), jnp.float32),
        ],
        scratch_types=[
            pltpu.VMEM((16,), jnp.float32),
            pltpu.VMEM((16,), jnp.int32),
            pltpu.VMEM((16,), jnp.float32),
        ],
        compiler_params=pltpu.CompilerParams(needs_layout_passes=False),
    )(_route_kernel)
    idx, w = route(glog.reshape(B * E))

    b1r = b1.reshape(E, 1, U1)
    b2r = b2.reshape(E, 1, U2)

    grid_spec = pltpu.PrefetchScalarGridSpec(
        num_scalar_prefetch=2,
        grid=(B, S // S_BLK, TOP_K),
        in_specs=[
            pl.BlockSpec((1, S_BLK, D), lambda b, s, k, idx, w: (b, s, 0)),
            pl.BlockSpec((1, D, U1),
                         lambda b, s, k, idx, w: (idx[2 * b + k], 0, 0)),
            pl.BlockSpec((1, 1, U1),
                         lambda b, s, k, idx, w: (idx[2 * b + k], 0, 0)),
            pl.BlockSpec((1, U1, U2),
                         lambda b, s, k, idx, w: (idx[2 * b + k], 0, 0)),
            pl.BlockSpec((1, 1, U2),
                         lambda b, s, k, idx, w: (idx[2 * b + k], 0, 0)),
        ],
        out_specs=pl.BlockSpec((1, S_BLK, U2),
                               lambda b, s, k, idx, w: (b, s, 0)),
    )

    out = pl.pallas_call(
        _ffn_kernel,
        grid_spec=grid_spec,
        out_shape=jax.ShapeDtypeStruct((B, S, U2), jnp.float32),
    )(idx, w, inputs, W1, b1r, W2, b2r)
    return out
